# baseline (speedup 1.0000x reference)
import functools

import jax
import jax.numpy as jnp
from jax import lax
from jax.experimental import pallas as pl
from jax.experimental.pallas import tpu as pltpu
from jax.experimental.pallas import tpu_sc as plsc

_ROWS = 16384
_COLS = 6
_NC = 2
_NS = 16
_NW = _NC * _NS
_ROWS_PER_W = _ROWS // _NW
_BLOCK = 32


@functools.partial(
    pl.kernel,
    out_type=jax.ShapeDtypeStruct((_ROWS, _COLS), jnp.float32),
    mesh=plsc.VectorSubcoreMesh(core_axis_name="c", subcore_axis_name="s"),
    scratch_types=[
        pltpu.VMEM((_BLOCK, _COLS), jnp.float32),
        pltpu.SemaphoreType.DMA,
    ],
)
def _pose_lookup(idx_hbm, table_hbm, out_hbm, rows_v, sem):
    del idx_hbm
    wid = lax.axis_index("s") * _NC + lax.axis_index("c")
    base = wid * _ROWS_PER_W

    seeds = [
        pltpu.async_copy(table_hbm, rows_v.at[pl.ds(j, 1)], sem)
        for j in range(_BLOCK)
    ]
    for c in seeds:
        c.wait()

    writes = [
        pltpu.async_copy(
            rows_v,
            out_hbm.at[pl.ds(base + j * _BLOCK, _BLOCK)],
            sem,
        )
        for j in range(_ROWS_PER_W // _BLOCK)
    ]
    for c in writes:
        c.wait()


def kernel(indices, table):
    return _pose_lookup(indices, table)


# speedup vs baseline: 6.8424x; 6.8424x over previous
import functools

import jax
import jax.numpy as jnp
from jax.experimental import pallas as pl
from jax.experimental.pallas import tpu as pltpu

_ROWS = 16384
_COLS = 6


def _broadcast_body(table_ref, out_ref):
    out_ref[...] = jnp.broadcast_to(table_ref[...], (_ROWS, _COLS))


@jax.jit
def _pose_lookup(table):
    return pl.pallas_call(
        _broadcast_body,
        out_shape=jax.ShapeDtypeStruct((_ROWS, _COLS), jnp.float32),
    )(table)


def kernel(indices, table):
    del indices
    return _pose_lookup(table)
